# baseline (speedup 1.0000x reference)
import functools

import jax
import jax.numpy as jnp
from jax import lax
from jax.experimental import pallas as pl
from jax.experimental.pallas import tpu as pltpu
from jax.experimental.pallas import tpu_sc as plsc

_NC = 2
_NS = 16
_NW = _NC * _NS
_D = 128
_C = 128
_NBUF = 7
_WLAG = 2


def _make_lookup(B):
    assert B % (_NW * _C) == 0
    bpw = B // _NW
    nchunk = bpw // _C

    mesh = plsc.VectorSubcoreMesh(core_axis_name="c", subcore_axis_name="s")

    @functools.partial(
        pl.kernel,
        mesh=mesh,
        out_type=jax.ShapeDtypeStruct((B, _D), jnp.float32),
        scratch_types=[
            pltpu.VMEM((bpw,), jnp.int32),
            pltpu.VMEM((_NBUF, _C, _D), jnp.float32),
            pltpu.VMEM_SHARED((2, _C, _D), jnp.float32),
        ] + [pltpu.SemaphoreType.DMA] * (_NBUF + 4),
    )
    def lookup(idx_hbm, tab_hbm, out_hbm, idx_v, rows_v, spm, *sems):
        sem_g = sems[:_NBUF]
        sem_x = sems[_NBUF:_NBUF + 2]
        sem_w = sems[_NBUF + 2:]
        cid = lax.axis_index("c")
        sid = lax.axis_index("s")
        wid = sid * _NC + cid
        base = wid * bpw
        pltpu.sync_copy(idx_hbm.at[pl.ds(base, bpw)], idx_v)

        def start_gather(j, b):
            pltpu.async_copy(
                tab_hbm.at[idx_v.at[pl.ds(j * _C, _C)]], rows_v.at[b],
                sem_g[b])

        def wait_gather(b):
            pltpu.make_async_copy(
                tab_hbm.at[pl.ds(0, _C)], rows_v.at[b], sem_g[b]).wait()

        def wait_xbar(q):
            pltpu.make_async_copy(
                rows_v.at[0], spm.at[q], sem_x[q]).wait()

        def start_hbm_write(j, q):
            pltpu.async_copy(
                spm.at[q], out_hbm.at[pl.ds(base + j * _C, _C)],
                sem_w[q])

        def wait_hbm_write(q):
            pltpu.make_async_copy(
                spm.at[q], out_hbm.at[pl.ds(0, _C)], sem_w[q]).wait()

        for b in range(_NBUF):
            start_gather(b, b)

        @pl.loop(0, nchunk, step=2 * _NBUF)
        def _iter(j0):
            for p in range(2 * _NBUF):
                j = j0 + p
                q = p % 2
                b = p % _NBUF

                @pl.when(j < nchunk)
                def _consume():
                    wait_gather(b)

                    @pl.when(j >= 2)
                    def _slot_free():
                        wait_hbm_write(q)

                    pltpu.async_copy(rows_v.at[b], spm.at[q], sem_x[q])

                    @pl.when(j >= 1)
                    def _drain_prev():
                        wait_xbar(1 - q)
                        start_hbm_write(j - 1, 1 - q)

                jn = j + _NBUF - _WLAG
                br = (b - _WLAG) % _NBUF

                @pl.when(jnp.logical_and(jn >= _NBUF, jn < nchunk))
                def _refill():
                    start_gather(jn, br)

        qlast = (nchunk - 1) % 2
        wait_xbar(qlast)
        start_hbm_write(nchunk - 1, qlast)
        wait_hbm_write(1 - qlast)
        wait_hbm_write(qlast)

    return lookup


def kernel(input, weights):
    b, s = input.shape
    flat_idx = input.reshape(b * s).astype(jnp.int32)
    out = _make_lookup(b * s)(flat_idx, weights)
    return out.reshape(b, s, weights.shape[1])
